# baseline (speedup 1.0000x reference)
import functools

import numpy as np
import jax
import jax.numpy as jnp
from jax import lax
from jax.experimental import pallas as pl
from jax.experimental.pallas import tpu as pltpu
from jax.experimental.pallas import tpu_sc as plsc

_DIM = 64
_B = 4096 * 200
_NW = 32
_PER_W = _B // _NW
_CHUNK = 128
_NCHUNK = _PER_W // _CHUNK

_BANDWIDTH = 2.0


def _gaussian_kernel_const() -> np.ndarray:
    t = np.arange(24, dtype=np.float32)
    ad = np.abs(t[None, :] - t[:, None])
    dist = np.minimum(ad, 24.0 - ad)
    return np.exp(-0.5 * (dist / _BANDWIDTH) ** 2).astype(np.float32)


_KMAT = _gaussian_kernel_const()


def _sc_gather(idx, table):
    mesh = plsc.VectorSubcoreMesh(core_axis_name="c", subcore_axis_name="s")

    @functools.partial(
        pl.kernel,
        mesh=mesh,
        out_type=jax.ShapeDtypeStruct((_B, _DIM), jnp.float32),
        scratch_types=[
            pltpu.VMEM((_NCHUNK, _CHUNK), jnp.int32),
            pltpu.VMEM((_CHUNK, _DIM), jnp.float32),
            pltpu.SemaphoreType.DMA,
        ],
        compiler_params=pltpu.CompilerParams(use_tc_tiling_on_sc=False),
    )
    def gather_kernel(idx_hbm, table_hbm, out_hbm, idx_v, rows_v, gsem):
        wid = lax.axis_index("s") * 2 + lax.axis_index("c")
        base = wid * _PER_W
        pltpu.sync_copy(idx_hbm.at[wid], idx_v)

        def body(j, carry):
            pltpu.async_copy(table_hbm.at[idx_v.at[j]], rows_v, gsem).wait()
            pltpu.sync_copy(rows_v, out_hbm.at[pl.ds(base + j * _CHUNK, _CHUNK)])
            return carry

        lax.fori_loop(0, _NCHUNK, body, 0)

    return gather_kernel(idx, table)


def _tc_small(user_table, time_table, kmat):
    def body(user_ref, time_ref, kmat_ref, user_out, time_out, smooth_out):
        i = pl.program_id(0)
        user_out[...] = user_ref[...]

        @pl.when(i == 0)
        def _():
            time_out[...] = time_ref[...]
            smooth_out[...] = jnp.dot(
                kmat_ref[...], time_ref[...], preferred_element_type=jnp.float32
            )

    n_users = user_table.shape[0]
    blk = n_users // 10
    return pl.pallas_call(
        body,
        grid=(10,),
        in_specs=[
            pl.BlockSpec((blk, _DIM), lambda i: (i, 0)),
            pl.BlockSpec((24, _DIM), lambda i: (0, 0)),
            pl.BlockSpec((24, 24), lambda i: (0, 0)),
        ],
        out_specs=[
            pl.BlockSpec((blk, _DIM), lambda i: (i, 0)),
            pl.BlockSpec((24, _DIM), lambda i: (0, 0)),
            pl.BlockSpec((24, _DIM), lambda i: (0, 0)),
        ],
        out_shape=[
            jax.ShapeDtypeStruct((n_users, _DIM), jnp.float32),
            jax.ShapeDtypeStruct((24, _DIM), jnp.float32),
            jax.ShapeDtypeStruct((24, _DIM), jnp.float32),
        ],
    )(user_table, time_table, kmat)


def kernel(location_x, loc_table, user_table, time_table):
    idx = location_x.astype(jnp.int32).reshape(_NW, _NCHUNK, _CHUNK)
    loc_flat = _sc_gather(idx, loc_table)
    loc_embedded = loc_flat.reshape(location_x.shape[0], location_x.shape[1], _DIM)
    user_out, time_out, smooth_out = _tc_small(
        user_table, time_table, jnp.asarray(_KMAT)
    )
    return (loc_embedded, time_out, smooth_out, user_out)


# speedup vs baseline: 1.3881x; 1.3881x over previous
import functools

import numpy as np
import jax
import jax.numpy as jnp
from jax import lax
from jax.experimental import pallas as pl
from jax.experimental.pallas import tpu as pltpu
from jax.experimental.pallas import tpu_sc as plsc

_DIM = 64
_NB = 4096
_NS = 200
_NW = 32
_BT = _NB // 128
_BANDWIDTH = 2.0


def _gaussian_kernel_const() -> np.ndarray:
    t = np.arange(24, dtype=np.float32)
    ad = np.abs(t[None, :] - t[:, None])
    dist = np.minimum(ad, 24.0 - ad)
    return np.exp(-0.5 * (dist / _BANDWIDTH) ** 2).astype(np.float32)


_KMAT = _gaussian_kernel_const()


def _sc_gather_fused(idx3, table):
    mesh = plsc.VectorSubcoreMesh(core_axis_name="c", subcore_axis_name="s")

    @functools.partial(
        pl.kernel,
        mesh=mesh,
        out_type=jax.ShapeDtypeStruct((_NS, 8, _BT, 8, 128), jnp.float32),
        scratch_types=[
            pltpu.VMEM((_NS, 128), jnp.int32),
            pltpu.VMEM((2, 128, _DIM), jnp.float32),
            pltpu.VMEM((2, 8, 8, 128), jnp.float32),
            pltpu.SemaphoreType.DMA,
            pltpu.SemaphoreType.DMA,
            pltpu.SemaphoreType.DMA,
            pltpu.SemaphoreType.DMA,
        ],
        compiler_params=pltpu.CompilerParams(
            use_tc_tiling_on_sc=False, needs_layout_passes=False
        ),
    )
    def gather_kernel(idx_hbm, table_hbm, out_hbm,
                      idxv, rowbuf, tbuf, gsem0, gsem1, wsem0, wsem1):
        w = lax.axis_index("s") * 2 + lax.axis_index("c")
        gsems = (gsem0, gsem1)
        wsems = (wsem0, wsem1)
        pltpu.sync_copy(idx_hbm.at[:, w], idxv)

        lvec = lax.iota(jnp.int32, 16)
        doffs = [(lvec + i) & 15 for i in range(16)]

        def gather_desc(s, b):
            return pltpu.make_async_copy(
                table_hbm.at[idxv.at[s]], rowbuf.at[b], gsems[b])

        def write_descs(s, b):
            return [
                pltpu.make_async_copy(
                    tbuf.at[b, dt], out_hbm.at[s, dt, w], wsems[b])
                for dt in range(8)
            ]

        def transpose(b):
            def blk(i, carry):
                kd = i >> 3
                kb = i & 7
                rows = kb * 16 + lvec
                for step in range(16):
                    d = kd * 16 + doffs[step]
                    v = plsc.load_gather(rowbuf.at[b], [rows, d])
                    plsc.store_scatter(
                        tbuf.at[b], [d >> 3, d & 7, kb * 16 + lvec], v)
                return carry

            lax.fori_loop(0, 32, blk, 0)

        gather_desc(0, 0).start()
        gather_desc(1, 1).start()

        def body(g, carry):
            for b in (0, 1):
                s = 2 * g + b
                gather_desc(s, b).wait()

                @pl.when(g >= 1)
                def _drain():
                    for d in write_descs(s - 2, b):
                        d.wait()

                transpose(b)
                for d in write_descs(s, b):
                    d.start()

                @pl.when(g < _NS // 2 - 1)
                def _next():
                    gather_desc(s + 2, b).start()

            return carry

        lax.fori_loop(0, _NS // 2, body, 0)
        for d in write_descs(_NS - 2, 0):
            d.wait()
        for d in write_descs(_NS - 1, 1):
            d.wait()

    return gather_kernel(idx3, table)


def _tc_small(user_t, time_table, kmat):
    nu = user_t.shape[1]
    blk = 8192
    grid = (nu + blk - 1) // blk

    def body(user_ref, time_ref, kmat_ref, user_out, time_out, smooth_out):
        i = pl.program_id(0)
        user_out[...] = user_ref[...]

        @pl.when(i == 0)
        def _():
            time_out[...] = time_ref[...]
            smooth_out[...] = jnp.dot(
                kmat_ref[...], time_ref[...], preferred_element_type=jnp.float32
            )

    return pl.pallas_call(
        body,
        grid=(grid,),
        in_specs=[
            pl.BlockSpec((_DIM, blk), lambda i: (0, i)),
            pl.BlockSpec((24, _DIM), lambda i: (0, 0)),
            pl.BlockSpec((24, 24), lambda i: (0, 0)),
        ],
        out_specs=[
            pl.BlockSpec((_DIM, blk), lambda i: (0, i)),
            pl.BlockSpec((24, _DIM), lambda i: (0, 0)),
            pl.BlockSpec((24, _DIM), lambda i: (0, 0)),
        ],
        out_shape=[
            jax.ShapeDtypeStruct((_DIM, nu), jnp.float32),
            jax.ShapeDtypeStruct((24, _DIM), jnp.float32),
            jax.ShapeDtypeStruct((24, _DIM), jnp.float32),
        ],
    )(user_t, time_table, kmat)


def kernel(location_x, loc_table, user_table, time_table):
    idx3 = jnp.transpose(location_x.astype(jnp.int32)).reshape(_NS, _BT, 128)
    x = _sc_gather_fused(idx3, loc_table)
    loc_embedded = x.transpose(2, 4, 0, 1, 3).reshape(_NB, _NS, _DIM)
    user_t_out, time_out, smooth_out = _tc_small(
        user_table.T, time_table, jnp.asarray(_KMAT)
    )
    return (loc_embedded, time_out, smooth_out, user_t_out.T)


# speedup vs baseline: 1.3948x; 1.0048x over previous
import functools

import numpy as np
import jax
import jax.numpy as jnp
from jax import lax
from jax.experimental import pallas as pl
from jax.experimental.pallas import tpu as pltpu
from jax.experimental.pallas import tpu_sc as plsc

_DIM = 64
_NB = 4096
_NS = 200
_NW = 32
_BT = _NB // 128
_BANDWIDTH = 2.0


def _gaussian_kernel_const() -> np.ndarray:
    t = np.arange(24, dtype=np.float32)
    ad = np.abs(t[None, :] - t[:, None])
    dist = np.minimum(ad, 24.0 - ad)
    return np.exp(-0.5 * (dist / _BANDWIDTH) ** 2).astype(np.float32)


_KMAT = _gaussian_kernel_const()


def _sc_gather_fused(idx3, table):
    mesh = plsc.VectorSubcoreMesh(core_axis_name="c", subcore_axis_name="s")

    @functools.partial(
        pl.kernel,
        mesh=mesh,
        out_type=jax.ShapeDtypeStruct((_NS, 8, _BT, 8, 128), jnp.float32),
        scratch_types=[
            pltpu.VMEM((_NS, 128), jnp.int32),
            pltpu.VMEM((4, 128, _DIM), jnp.float32),
            pltpu.VMEM((2, 8, 8, 128), jnp.float32),
            pltpu.SemaphoreType.DMA,
            pltpu.SemaphoreType.DMA,
            pltpu.SemaphoreType.DMA,
            pltpu.SemaphoreType.DMA,
            pltpu.SemaphoreType.DMA,
            pltpu.SemaphoreType.DMA,
        ],
        compiler_params=pltpu.CompilerParams(
            use_tc_tiling_on_sc=False, needs_layout_passes=False
        ),
    )
    def gather_kernel(idx_hbm, table_hbm, out_hbm, idxv, rowbuf, tbuf,
                      gsem0, gsem1, gsem2, gsem3, wsem0, wsem1):
        w = lax.axis_index("s") * 2 + lax.axis_index("c")
        gsems = (gsem0, gsem1, gsem2, gsem3)
        wsems = (wsem0, wsem1)
        pltpu.sync_copy(idx_hbm.at[:, w], idxv)

        lvec = lax.iota(jnp.int32, 16)
        doffs = [(lvec + i) & 15 for i in range(16)]

        def gather_desc(s, b):
            return pltpu.make_async_copy(
                table_hbm.at[idxv.at[s]], rowbuf.at[b], gsems[b])

        def write_desc(s, b):
            return pltpu.make_async_copy(
                tbuf.at[b], out_hbm.at[s, :, w], wsems[b])

        def transpose(b, b2):
            def blk(i, carry):
                kd = i >> 3
                kb = i & 7
                rows = kb * 16 + lvec
                for step in range(16):
                    d = kd * 16 + doffs[step]
                    v = plsc.load_gather(rowbuf.at[b], [rows, d])
                    plsc.store_scatter(
                        tbuf.at[b2], [d >> 3, d & 7, kb * 16 + lvec], v)
                return carry

            lax.fori_loop(0, 32, blk, 0)

        for s0 in (0, 1, 2, 3):
            gather_desc(s0, s0).start()

        def body(g, carry):
            for b in (0, 1, 2, 3):
                s = 4 * g + b
                tb = b % 2
                gather_desc(s, b).wait()

                if b >= 2:
                    write_desc(s - 2, tb).wait()
                else:
                    @pl.when(g >= 1)
                    def _drain():
                        write_desc(s - 2, tb).wait()

                transpose(b, tb)

                @pl.when(g < _NS // 4 - 1)
                def _next():
                    gather_desc(s + 4, b).start()

                write_desc(s, tb).start()

            return carry

        lax.fori_loop(0, _NS // 4, body, 0)
        write_desc(_NS - 2, 0).wait()
        write_desc(_NS - 1, 1).wait()

    return gather_kernel(idx3, table)


def _tc_small(user_t, time_table, kmat):
    nu = user_t.shape[1]
    blk = 8192
    grid = (nu + blk - 1) // blk

    def body(user_ref, time_ref, kmat_ref, user_out, time_out, smooth_out):
        i = pl.program_id(0)
        user_out[...] = user_ref[...]

        @pl.when(i == 0)
        def _():
            time_out[...] = time_ref[...]
            smooth_out[...] = jnp.dot(
                kmat_ref[...], time_ref[...], preferred_element_type=jnp.float32
            )

    return pl.pallas_call(
        body,
        grid=(grid,),
        in_specs=[
            pl.BlockSpec((_DIM, blk), lambda i: (0, i)),
            pl.BlockSpec((24, _DIM), lambda i: (0, 0)),
            pl.BlockSpec((24, 24), lambda i: (0, 0)),
        ],
        out_specs=[
            pl.BlockSpec((_DIM, blk), lambda i: (0, i)),
            pl.BlockSpec((24, _DIM), lambda i: (0, 0)),
            pl.BlockSpec((24, _DIM), lambda i: (0, 0)),
        ],
        out_shape=[
            jax.ShapeDtypeStruct((_DIM, nu), jnp.float32),
            jax.ShapeDtypeStruct((24, _DIM), jnp.float32),
            jax.ShapeDtypeStruct((24, _DIM), jnp.float32),
        ],
    )(user_t, time_table, kmat)


def kernel(location_x, loc_table, user_table, time_table):
    idx3 = jnp.transpose(location_x.astype(jnp.int32)).reshape(_NS, _BT, 128)
    x = _sc_gather_fused(idx3, loc_table)
    loc_embedded = x.transpose(2, 4, 0, 1, 3).reshape(_NB, _NS, _DIM)
    user_t_out, time_out, smooth_out = _tc_small(
        user_table.T, time_table, jnp.asarray(_KMAT)
    )
    return (loc_embedded, time_out, smooth_out, user_t_out.T)
